# baseline (speedup 1.0000x reference)
import functools

import jax
import jax.numpy as jnp
from jax import lax
from jax.experimental import pallas as pl
from jax.experimental.pallas import tpu as pltpu
from jax.experimental.pallas import tpu_sc as plsc

_NC = 2
_NS = 16
_NW = _NC * _NS
_L = 16
_CHUNK = 32768


def _make_sc_kernel(total, pe_span):
    per_worker = total // _NW
    n_chunks = per_worker // _CHUNK
    mesh = plsc.VectorSubcoreMesh(core_axis_name="c", subcore_axis_name="s")

    @functools.partial(
        pl.kernel,
        mesh=mesh,
        out_type=jax.ShapeDtypeStruct((total,), jnp.float32),
        scratch_types=[
            pltpu.VMEM((_CHUNK,), jnp.float32),
            pltpu.VMEM((_CHUNK,), jnp.float32),
        ],
    )
    def sc_add(x_hbm, pe_hbm, out_hbm, xbuf, pebuf):
        wid = lax.axis_index("s") * _NC + lax.axis_index("c")
        base = wid * per_worker
        pe_base = lax.rem(base, pe_span)
        for c in range(n_chunks):
            xoff = base + c * _CHUNK
            poff = pe_base + c * _CHUNK
            pltpu.sync_copy(x_hbm.at[pl.ds(xoff, _CHUNK)], xbuf)
            pltpu.sync_copy(pe_hbm.at[pl.ds(poff, _CHUNK)], pebuf)

            def body(i, _):
                for u in range(4):
                    off = (i * 4 + u) * _L
                    xbuf[pl.ds(off, _L)] = xbuf[pl.ds(off, _L)] + pebuf[
                        pl.ds(off, _L)
                    ]
                return 0

            lax.fori_loop(0, _CHUNK // (4 * _L), body, 0)
            pltpu.sync_copy(xbuf, out_hbm.at[pl.ds(xoff, _CHUNK)])

    return sc_add


def kernel(x, pos_table):
    B, S, D = x.shape
    total = B * S * D
    pe_span = S * D
    sc_add = _make_sc_kernel(total, pe_span)
    out = sc_add(x.reshape(total), pos_table.reshape(-1))
    return out.reshape(B, S, D)


# speedup vs baseline: 1.2023x; 1.2023x over previous
import functools

import jax
import jax.numpy as jnp
from jax import lax
from jax.experimental import pallas as pl
from jax.experimental.pallas import tpu as pltpu
from jax.experimental.pallas import tpu_sc as plsc

_NC = 2
_NS = 16
_NW = _NC * _NS
_L = 16
_CHUNK = 8192
_NBUF = 4
_UNROLL = 8


def _make_sc_kernel(total, pe_span):
    per_worker = total // _NW
    n_chunks = per_worker // _CHUNK
    mesh = plsc.VectorSubcoreMesh(core_axis_name="c", subcore_axis_name="s")

    scratch = (
        [pltpu.VMEM((_CHUNK,), jnp.float32) for _ in range(2 * _NBUF)]
        + [pltpu.SemaphoreType.DMA for _ in range(3 * _NBUF)]
    )

    @functools.partial(
        pl.kernel,
        mesh=mesh,
        out_type=jax.ShapeDtypeStruct((total,), jnp.float32),
        scratch_types=scratch,
    )
    def sc_add(x_hbm, pe_hbm, out_hbm, *bufs_and_sems):
        xbufs = bufs_and_sems[:_NBUF]
        pebufs = bufs_and_sems[_NBUF : 2 * _NBUF]
        lsemx = bufs_and_sems[2 * _NBUF : 3 * _NBUF]
        lsemp = bufs_and_sems[3 * _NBUF : 4 * _NBUF]
        ssem = bufs_and_sems[4 * _NBUF : 5 * _NBUF]

        wid = lax.axis_index("s") * _NC + lax.axis_index("c")
        base = wid * per_worker
        pe_base = lax.rem(base, pe_span)

        load_h = [None] * _NBUF
        store_h = [None] * _NBUF

        def start_load(c):
            b = c % _NBUF
            xoff = base + c * _CHUNK
            poff = pe_base + c * _CHUNK
            load_h[b] = (
                pltpu.async_copy(x_hbm.at[pl.ds(xoff, _CHUNK)], xbufs[b], lsemx[b]),
                pltpu.async_copy(pe_hbm.at[pl.ds(poff, _CHUNK)], pebufs[b], lsemp[b]),
            )

        start_load(0)
        if n_chunks > 1:
            start_load(1)

        for c in range(n_chunks):
            b = c % _NBUF
            hx, hp = load_h[b]
            hx.wait()
            hp.wait()

            nxt = c + 2
            if nxt < n_chunks:
                nb = nxt % _NBUF
                if store_h[nb] is not None:
                    store_h[nb].wait()
                    store_h[nb] = None
                start_load(nxt)

            xbuf, pebuf = xbufs[b], pebufs[b]

            def body(i, _):
                for u in range(_UNROLL):
                    off = (i * _UNROLL + u) * _L
                    xbuf[pl.ds(off, _L)] = (
                        xbuf[pl.ds(off, _L)] + pebuf[pl.ds(off, _L)]
                    )
                return 0

            lax.fori_loop(0, _CHUNK // (_UNROLL * _L), body, 0)

            xoff = base + c * _CHUNK
            store_h[b] = pltpu.async_copy(
                xbuf, out_hbm.at[pl.ds(xoff, _CHUNK)], ssem[b]
            )

        for b in range(_NBUF):
            if store_h[b] is not None:
                store_h[b].wait()

    return sc_add


def kernel(x, pos_table):
    B, S, D = x.shape
    total = B * S * D
    pe_span = S * D
    sc_add = _make_sc_kernel(total, pe_span)
    out = sc_add(x.reshape(total), pos_table.reshape(-1))
    return out.reshape(B, S, D)


# speedup vs baseline: 6.2634x; 5.2093x over previous
import jax
import jax.numpy as jnp
from jax.experimental import pallas as pl
from jax.experimental.pallas import tpu as pltpu

_S_BLK = 2048


def _add_kernel(x_ref, pe_ref, o_ref):
    o_ref[...] = x_ref[...] + pe_ref[...]


def kernel(x, pos_table):
    B, S, D = x.shape
    grid = (S // _S_BLK, B)
    return pl.pallas_call(
        _add_kernel,
        grid=grid,
        in_specs=[
            pl.BlockSpec((1, _S_BLK, D), lambda s, b: (b, s, 0)),
            pl.BlockSpec((_S_BLK, D), lambda s, b: (s, 0)),
        ],
        out_specs=pl.BlockSpec((1, _S_BLK, D), lambda s, b: (b, s, 0)),
        out_shape=jax.ShapeDtypeStruct((B, S, D), x.dtype),
        compiler_params=pltpu.CompilerParams(
            dimension_semantics=("parallel", "parallel"),
        ),
    )(x, pos_table)
